# Initial kernel scaffold; baseline (speedup 1.0000x reference)
import jax
import jax.numpy as jnp
from jax.experimental import pallas as pl


def kernel(input, cloud_len_list, voxel_num, W1, W2, W3, W4, W5, g5, b5, Wl1, g6, b6, Wl2, bl2, g7, b7, Wl3, bl3, W6, gc6, bc6, W7, gc7, bc7, W8, gc8, bc8, Wl4, Wl5, bl5):
    raise NotImplementedError("write your pallas kernel here")


# speedup vs baseline: 3.9791x; 3.9791x over previous
import functools
import jax
import jax.numpy as jnp
from jax import lax
from jax.experimental import pallas as pl
from jax.experimental.pallas import tpu as pltpu
from jax.experimental.pallas import tpu_sc as plsc

_N = 512
_B = 32
_K = 16
_VK = 8
_P = 128
_NEG = -1e30


def _lr(t):
    return jnp.where(t >= 0, t, 0.2 * t)


def _bdot(a, b):
    return lax.dot_general(a.astype(jnp.bfloat16), b.astype(jnp.bfloat16),
                           (((1,), (1,)), ((), ())),
                           preferred_element_type=jnp.float32)


def _pd_topk(xt, xx, n, k, idx_ref, base):
    xtx = _bdot(xt, xt)
    pd = 2.0 * xtx - xx[:, None] - xx[None, :]
    pd = pd * jnp.where(xx > 0, 1.0, 1e7)[None, :]
    cols = lax.broadcasted_iota(jnp.int32, (n, n), 1)
    for r in range(k):
        mx = jnp.max(pd, axis=1, keepdims=True)
        am = jnp.min(jnp.where(pd == mx, cols, n), axis=1)
        idx_ref[r, :] = am + base
        pd = jnp.where(cols == am[:, None], _NEG, pd)


def _edge_conv(x, g_ref, wcat, c):
    xv = x[:, :c]
    xb = xv.astype(jnp.bfloat16)
    acc = None
    for k in range(_K):
        gk = g_ref[pl.ds(k * _N, _N), pl.ds(0, c)]
        e = jnp.concatenate([(gk - xv).astype(jnp.bfloat16), xb], axis=1)
        ck = lax.dot_general(e, wcat, (((1,), (1,)), ((), ())),
                             preferred_element_type=jnp.float32)
        acc = ck if acc is None else jnp.maximum(acc, ck)
    return _lr(acc)


def _topk_body(x_ref, xx_ref, idx_ref):
    _pd_topk(x_ref[...], xx_ref[0, :], _N, _K, idx_ref, pl.program_id(0) * _N)


def _topk_call(x, xx):
    B, N, P = x.shape
    return pl.pallas_call(
        _topk_body,
        grid=(B,),
        in_specs=[
            pl.BlockSpec((None, N, P), lambda b: (b, 0, 0)),
            pl.BlockSpec((None, 1, N), lambda b: (b, 0, 0)),
        ],
        out_specs=pl.BlockSpec((None, _K, N), lambda b: (b, 0, 0)),
        out_shape=jax.ShapeDtypeStruct((B, _K, N), jnp.int32),
    )(x, xx.reshape(B, 1, N))


def _conv_body(c, x_ref, g_ref, wc_ref, xn_ref):
    xn_ref[...] = _edge_conv(x_ref[...], g_ref, wc_ref[...], c)


def _conv_call(x, g, wc):
    B, N, P = x.shape
    c = wc.shape[1] // 2
    return pl.pallas_call(
        functools.partial(_conv_body, c),
        grid=(B,),
        in_specs=[
            pl.BlockSpec((None, N, P), lambda b: (b, 0, 0)),
            pl.BlockSpec((None, _K * N, P), lambda b: (b, 0, 0)),
            pl.BlockSpec(wc.shape, lambda b: (0, 0)),
        ],
        out_specs=pl.BlockSpec((None, N, P), lambda b: (b, 0, 0)),
        out_shape=jax.ShapeDtypeStruct((B, N, P), jnp.float32),
    )(x, g, wc)


_CH_ROWS = 512
_GSZ = 128


@jax.jit
def _gather_rows(xflat, idxg):
    NROW = xflat.shape[0]
    n_g = _CH_ROWS // _GSZ
    n_chunks = (_K * _N) // _CH_ROWS
    mesh = plsc.VectorSubcoreMesh(core_axis_name="c", subcore_axis_name="s")

    @functools.partial(
        pl.kernel,
        mesh=mesh,
        out_type=jax.ShapeDtypeStruct((_B * _K * _N, _P), jnp.float32),
        scratch_types=[
            pltpu.VMEM((_CH_ROWS,), jnp.int32),
            pltpu.VMEM((_CH_ROWS, _P), jnp.float32),
            pltpu.SemaphoreType.DMA,
        ],
    )
    def gm(x_hbm, idx_hbm, out_hbm, idx_v, rows_v, sem):
        wid = lax.axis_index("s") * 2 + lax.axis_index("c")

        def chunk_body(c, _):
            base = wid * (_K * _N) + c * _CH_ROWS
            pltpu.sync_copy(idx_hbm.at[pl.ds(base, _CH_ROWS)], idx_v)
            cps = []
            for g in range(n_g):
                cps.append(pltpu.async_copy(
                    x_hbm.at[idx_v.at[pl.ds(g * _GSZ, _GSZ)]],
                    rows_v.at[pl.ds(g * _GSZ, _GSZ)], sem))
            for cp in cps:
                cp.wait()
            pltpu.sync_copy(rows_v, out_hbm.at[pl.ds(base, _CH_ROWS)])
            return 0

        lax.fori_loop(0, n_chunks, chunk_body, 0)

    return gm(xflat, idxg)


def _s5a_body(x3_ref, g3_ref, w4_ref, x1_ref, x2_ref, w5_ref,
              z_ref, ss_ref, sq_ref):
    x4 = _edge_conv(x3_ref[...], g3_ref, w4_ref[...], 64)
    xcat = jnp.concatenate(
        [x1_ref[...][:, :32], x2_ref[...][:, :32], x3_ref[...][:, :64], x4],
        axis=1).astype(jnp.bfloat16)
    z = lax.dot_general(xcat, w5_ref[...], (((1,), (1,)), ((), ())),
                        preferred_element_type=jnp.float32)
    z_ref[...] = z

    @pl.when(pl.program_id(0) == 0)
    def _():
        ss_ref[...] = jnp.zeros_like(ss_ref)
        sq_ref[...] = jnp.zeros_like(sq_ref)

    ss_ref[...] += jnp.sum(z, axis=0)
    sq_ref[...] += jnp.sum(z * z, axis=0)


def _s5a_call(x3, g3, w4, x1, x2, w5):
    B, N, P = x3.shape
    E = w5.shape[0]
    return pl.pallas_call(
        _s5a_body,
        grid=(B,),
        in_specs=[
            pl.BlockSpec((None, N, P), lambda b: (b, 0, 0)),
            pl.BlockSpec((None, _K * N, P), lambda b: (b, 0, 0)),
            pl.BlockSpec(w4.shape, lambda b: (0, 0)),
            pl.BlockSpec((None, N, P), lambda b: (b, 0, 0)),
            pl.BlockSpec((None, N, P), lambda b: (b, 0, 0)),
            pl.BlockSpec((E, 2 * P), lambda b: (0, 0)),
        ],
        out_specs=[
            pl.BlockSpec((None, N, E), lambda b: (b, 0, 0)),
            pl.BlockSpec((E,), lambda b: (0,)),
            pl.BlockSpec((E,), lambda b: (0,)),
        ],
        out_shape=[
            jax.ShapeDtypeStruct((B, N, E), jnp.float32),
            jax.ShapeDtypeStruct((E,), jnp.float32),
            jax.ShapeDtypeStruct((E,), jnp.float32),
        ],
        compiler_params=pltpu.CompilerParams(
            dimension_semantics=("arbitrary",)),
    )(x3, g3, w4, x1, x2, w5)


def _s5b_body(z_ref, sc_ref, sh_ref, f_ref):
    u = _lr(z_ref[...] * sc_ref[...][None, :] + sh_ref[...][None, :])
    fmax = jnp.max(u, axis=0)
    fmean = jnp.sum(u, axis=0) * (1.0 / _N)
    f_ref[...] = jnp.concatenate([fmax, fmean])[None, :]


def _s5b_call(z, scale, shift):
    B, N, E = z.shape
    return pl.pallas_call(
        _s5b_body,
        grid=(B,),
        in_specs=[
            pl.BlockSpec((None, N, E), lambda b: (b, 0, 0)),
            pl.BlockSpec((E,), lambda b: (0,)),
            pl.BlockSpec((E,), lambda b: (0,)),
        ],
        out_specs=pl.BlockSpec((None, 1, 2 * E), lambda b: (b, 0, 0)),
        out_shape=jax.ShapeDtypeStruct((B, 1, 2 * E), jnp.float32),
    )(z, scale, shift).reshape(B, 2 * E)


def _bn_rows(h, g, b):
    m = jnp.mean(h, axis=0)
    d = h - m[None, :]
    v = jnp.mean(d * d, axis=0)
    return g[None, :] * d / jnp.sqrt(v + 1e-5)[None, :] + b[None, :]


def _mm(a, b):
    return lax.dot_general(a.astype(jnp.bfloat16), b.astype(jnp.bfloat16),
                           (((1,), (0,)), ((), ())),
                           preferred_element_type=jnp.float32)


def _s6_body(f_ref, wl1_ref, g6_ref, b6_ref, wl2_ref, bl2_ref, g7_ref,
             b7_ref, wl3_ref, bl3_ref, o_ref):
    h = _mm(f_ref[...], wl1_ref[...])
    h = _lr(_bn_rows(h, g6_ref[...], b6_ref[...]))
    h = _mm(h, wl2_ref[...]) + bl2_ref[...][None, :]
    h = _lr(_bn_rows(h, g7_ref[...], b7_ref[...]))
    o_ref[...] = _mm(h, wl3_ref[...]) + bl3_ref[...][None, :]


def _s6_call(feat, wl1, g6, b6, wl2, bl2, g7, b7, wl3, bl3):
    B = feat.shape[0]
    C = wl3.shape[1]
    return pl.pallas_call(
        _s6_body,
        out_shape=jax.ShapeDtypeStruct((B, C), jnp.float32),
    )(feat, wl1, g6, b6, wl2, bl2, g7, b7, wl3, bl3)


def _vox_edge(xt, wc, g, b, nn, k):
    cols = lax.broadcasted_iota(jnp.int32, (nn, nn), 1)
    xtx = _bdot(xt, xt)
    xx = jnp.sum(xt * xt, axis=1)
    pd = 2.0 * xtx - xx[:, None] - xx[None, :]
    pd = pd * jnp.where(xx > 0, 1.0, 1e7)[None, :]
    O = wc.shape[0]
    cmax = jnp.full((nn, O), _NEG, jnp.float32)
    cs = []
    for r in range(k):
        mx = jnp.max(pd, axis=1, keepdims=True)
        am = jnp.min(jnp.where(pd == mx, cols, nn), axis=1)
        pd = jnp.where(cols == am[:, None], _NEG, pd)
        p = (am[:, None] == cols).astype(jnp.float32)
        gth = jnp.dot(p, xt, preferred_element_type=jnp.float32,
                      precision=lax.Precision.HIGHEST)
        e = jnp.concatenate([gth - xt, xt], axis=1)
        c = _bdot(e, wc)
        cs.append(c)
        cmax = jnp.maximum(cmax, c)
    mm = sum(jnp.sum(c, axis=0) for c in cs) * (1.0 / (nn * k))
    vv = sum(jnp.sum((c - mm[None, :]) ** 2, axis=0) for c in cs) * (1.0 / (nn * k))
    return _lr(g[None, :] * (cmax - mm[None, :]) / jnp.sqrt(vv + 1e-5)[None, :] + b[None, :])


def _s7_body(v_ref, w6_ref, gc6_ref, bc6_ref, w7_ref, gc7_ref, bc7_ref,
             w8_ref, gc8_ref, bc8_ref, wl4_ref, wl5_ref, bl5_ref, o_ref):
    v = v_ref[...]
    nn = v.shape[0]
    x1 = _vox_edge(v, w6_ref[...], gc6_ref[...], bc6_ref[...], nn, _VK)
    x2 = _vox_edge(x1, w7_ref[...], gc7_ref[...], bc7_ref[...], nn, _VK)
    cat = jnp.concatenate([x1, x2], axis=1)
    c3 = _bdot(cat, w8_ref[...])
    u = _lr(_bn_rows(c3, gc8_ref[...], bc8_ref[...]))
    ff = jnp.concatenate([jnp.max(u, axis=0), jnp.sum(u, axis=0) * (1.0 / nn)])
    h = _lr(_mm(ff[None, :], wl4_ref[...]))
    o_ref[...] = _mm(h, wl5_ref[...]) + bl5_ref[...][None, :]


def _s7_call(v, w6, gc6, bc6, w7, gc7, bc7, w8, gc8, bc8, wl4, wl5, bl5):
    return pl.pallas_call(
        _s7_body,
        out_shape=jax.ShapeDtypeStruct((1, wl5.shape[1]), jnp.float32),
    )(v, w6, gc6, bc6, w7, gc7, bc7, w8, gc8, bc8, wl4, wl5, bl5)


def _pad_w(w):
    o = w.shape[0]
    return jnp.pad(w, ((0, _P - o), (0, 0))).astype(jnp.bfloat16)


def kernel(input, cloud_len_list, voxel_num, W1, W2, W3, W4, W5, g5, b5,
           Wl1, g6, b6, Wl2, bl2, g7, b7, Wl3, bl3, W6, gc6, bc6, W7, gc7,
           bc7, W8, gc8, bc8, Wl4, Wl5, bl5):
    B, N, C0 = input.shape
    x0 = jnp.pad(input, ((0, 0), (0, 0), (0, _P - C0)))

    def gather(x, idx):
        g = _gather_rows(x.reshape(B * N, _P), idx.reshape(B * _K * _N))
        return g.reshape(B, _K * N, _P)

    rn = lambda t: jnp.sum(t * t, axis=2)
    idx1 = _topk_call(x0, rn(x0))
    g0 = gather(x0, idx1)
    x1 = _conv_call(x0, g0, _pad_w(W1))
    idx2 = _topk_call(x1, rn(x1))
    g1 = gather(x1, idx2)
    x2 = _conv_call(x1, g1, _pad_w(W2))
    idx3 = _topk_call(x2, rn(x2))
    g2 = gather(x2, idx3)
    x3 = _conv_call(x2, g2, _pad_w(W3))
    idx4 = _topk_call(x3, rn(x3))
    g3 = gather(x3, idx4)

    z, ssum, ssq = _s5a_call(x3, g3, _pad_w(W4), x1, x2,
                             W5.astype(jnp.bfloat16))
    cnt = B * N
    mean = ssum / cnt
    var = ssq / cnt - mean * mean
    scale = g5 / jnp.sqrt(var + 1e-5)
    shift = b5 - mean * scale
    feat = _s5b_call(z, scale, shift)

    v = _s6_call(feat, Wl1, g6, b6, Wl2, bl2, g7, b7, Wl3, bl3)
    return _s7_call(v, W6, gc6, bc6, W7, gc7, bc7, W8, gc8, bc8, Wl4, Wl5, bl5)


# speedup vs baseline: 4.0443x; 1.0164x over previous
import functools
import jax
import jax.numpy as jnp
from jax import lax
from jax.experimental import pallas as pl
from jax.experimental.pallas import tpu as pltpu
from jax.experimental.pallas import tpu_sc as plsc

_N = 512
_B = 32
_K = 16
_VK = 8
_P = 128
_NEG = -1e30


def _lr(t):
    return jnp.where(t >= 0, t, 0.2 * t)


def _bdot(a, b):
    return lax.dot_general(a.astype(jnp.bfloat16), b.astype(jnp.bfloat16),
                           (((1,), (1,)), ((), ())),
                           preferred_element_type=jnp.float32)


def _pd_topk(xt, xx, n, k, idx_ref, base):
    xtx = _bdot(xt, xt)
    pd = 2.0 * xtx - xx[:, None] - xx[None, :]
    pd = pd * jnp.where(xx > 0, 1.0, 1e7)[None, :]
    cols = lax.broadcasted_iota(jnp.int32, (n, n), 1)
    for r in range(k):
        mx = jnp.max(pd, axis=1, keepdims=True)
        am = jnp.min(jnp.where(pd == mx, cols, n), axis=1)
        idx_ref[r, :] = am + base
        pd = jnp.where(cols == am[:, None], _NEG, pd)


def _edge_conv(x, g_ref, wcat, c):
    xv = x[:, :c]
    xb = xv.astype(jnp.bfloat16)
    acc = None
    for k in range(_K):
        gk = g_ref[pl.ds(k * _N, _N), pl.ds(0, c)]
        e = jnp.concatenate([(gk - xv).astype(jnp.bfloat16), xb], axis=1)
        ck = lax.dot_general(e, wcat, (((1,), (1,)), ((), ())),
                             preferred_element_type=jnp.float32)
        acc = ck if acc is None else jnp.maximum(acc, ck)
    return _lr(acc)


def _topk_body(x_ref, xx_ref, idx_ref):
    _pd_topk(x_ref[...], xx_ref[0, :], _N, _K, idx_ref, pl.program_id(0) * _N)


def _topk_call(x, xx):
    B, N, P = x.shape
    return pl.pallas_call(
        _topk_body,
        grid=(B,),
        in_specs=[
            pl.BlockSpec((None, N, P), lambda b: (b, 0, 0)),
            pl.BlockSpec((None, 1, N), lambda b: (b, 0, 0)),
        ],
        out_specs=pl.BlockSpec((None, _K, N), lambda b: (b, 0, 0)),
        out_shape=jax.ShapeDtypeStruct((B, _K, N), jnp.int32),
    )(x, xx.reshape(B, 1, N))


def _conv_body(c, x_ref, g_ref, wc_ref, xn_ref):
    xn_ref[...] = _edge_conv(x_ref[...], g_ref, wc_ref[...], c)


def _conv_call(x, g, wc):
    B, N, P = x.shape
    c = wc.shape[1] // 2
    return pl.pallas_call(
        functools.partial(_conv_body, c),
        grid=(B,),
        in_specs=[
            pl.BlockSpec((None, N, P), lambda b: (b, 0, 0)),
            pl.BlockSpec((None, _K * N, P), lambda b: (b, 0, 0)),
            pl.BlockSpec(wc.shape, lambda b: (0, 0)),
        ],
        out_specs=pl.BlockSpec((None, N, P), lambda b: (b, 0, 0)),
        out_shape=jax.ShapeDtypeStruct((B, N, P), jnp.float32),
    )(x, g, wc)


_CH_ROWS = 256
_GSZ = 128
_NBUF = 3


@jax.jit
def _gather_rows(xflat, idxg):
    n_g = _CH_ROWS // _GSZ
    n_chunks = (_K * _N) // _CH_ROWS
    mesh = plsc.VectorSubcoreMesh(core_axis_name="c", subcore_axis_name="s")

    @functools.partial(
        pl.kernel,
        mesh=mesh,
        out_type=jax.ShapeDtypeStruct((_B * _K * _N, _P), jnp.float32),
        scratch_types=[pltpu.VMEM((_CH_ROWS,), jnp.int32)] * _NBUF
        + [pltpu.VMEM((_CH_ROWS, _P), jnp.float32)] * _NBUF
        + [pltpu.SemaphoreType.DMA, pltpu.SemaphoreType.DMA],
    )
    def gm(x_hbm, idx_hbm, out_hbm, *scr):
        idx_v = scr[:_NBUF]
        rows_v = scr[_NBUF:2 * _NBUF]
        sg, so = scr[2 * _NBUF], scr[2 * _NBUF + 1]
        wid = lax.axis_index("s") * 2 + lax.axis_index("c")
        base0 = wid * (_K * _N)
        cps_g = [None] * _NBUF
        cps_o = [None] * _NBUF
        for c in range(n_chunks + 2):
            if c < n_chunks:
                buf = c % _NBUF
                if cps_o[buf] is not None:
                    cps_o[buf].wait()
                    cps_o[buf] = None
                base = base0 + c * _CH_ROWS
                pltpu.sync_copy(idx_hbm.at[pl.ds(base, _CH_ROWS)], idx_v[buf])
                cps_g[buf] = [
                    pltpu.async_copy(
                        x_hbm.at[idx_v[buf].at[pl.ds(g * _GSZ, _GSZ)]],
                        rows_v[buf].at[pl.ds(g * _GSZ, _GSZ)], sg)
                    for g in range(n_g)]
            if c >= 2:
                cc = c - 2
                buf2 = cc % _NBUF
                for cp in cps_g[buf2]:
                    cp.wait()
                cps_g[buf2] = []
                cps_o[buf2] = pltpu.async_copy(
                    rows_v[buf2],
                    out_hbm.at[pl.ds(base0 + cc * _CH_ROWS, _CH_ROWS)], so)
        for b in range(_NBUF):
            if cps_o[b] is not None:
                cps_o[b].wait()

    return gm(xflat, idxg)


def _s5a_body(x3_ref, g3_ref, w4_ref, x1_ref, x2_ref, w5_ref,
              z_ref, ss_ref, sq_ref):
    x4 = _edge_conv(x3_ref[...], g3_ref, w4_ref[...], 64)
    xcat = jnp.concatenate(
        [x1_ref[...][:, :32], x2_ref[...][:, :32], x3_ref[...][:, :64], x4],
        axis=1).astype(jnp.bfloat16)
    z = lax.dot_general(xcat, w5_ref[...], (((1,), (1,)), ((), ())),
                        preferred_element_type=jnp.float32)
    z_ref[...] = z

    @pl.when(pl.program_id(0) == 0)
    def _():
        ss_ref[...] = jnp.zeros_like(ss_ref)
        sq_ref[...] = jnp.zeros_like(sq_ref)

    ss_ref[...] += jnp.sum(z, axis=0)
    sq_ref[...] += jnp.sum(z * z, axis=0)


def _s5a_call(x3, g3, w4, x1, x2, w5):
    B, N, P = x3.shape
    E = w5.shape[0]
    return pl.pallas_call(
        _s5a_body,
        grid=(B,),
        in_specs=[
            pl.BlockSpec((None, N, P), lambda b: (b, 0, 0)),
            pl.BlockSpec((None, _K * N, P), lambda b: (b, 0, 0)),
            pl.BlockSpec(w4.shape, lambda b: (0, 0)),
            pl.BlockSpec((None, N, P), lambda b: (b, 0, 0)),
            pl.BlockSpec((None, N, P), lambda b: (b, 0, 0)),
            pl.BlockSpec((E, 2 * P), lambda b: (0, 0)),
        ],
        out_specs=[
            pl.BlockSpec((None, N, E), lambda b: (b, 0, 0)),
            pl.BlockSpec((E,), lambda b: (0,)),
            pl.BlockSpec((E,), lambda b: (0,)),
        ],
        out_shape=[
            jax.ShapeDtypeStruct((B, N, E), jnp.float32),
            jax.ShapeDtypeStruct((E,), jnp.float32),
            jax.ShapeDtypeStruct((E,), jnp.float32),
        ],
        compiler_params=pltpu.CompilerParams(
            dimension_semantics=("arbitrary",)),
    )(x3, g3, w4, x1, x2, w5)


def _s5b_body(z_ref, sc_ref, sh_ref, f_ref):
    u = _lr(z_ref[...] * sc_ref[...][None, :] + sh_ref[...][None, :])
    fmax = jnp.max(u, axis=0)
    fmean = jnp.sum(u, axis=0) * (1.0 / _N)
    f_ref[...] = jnp.concatenate([fmax, fmean])[None, :]


def _s5b_call(z, scale, shift):
    B, N, E = z.shape
    return pl.pallas_call(
        _s5b_body,
        grid=(B,),
        in_specs=[
            pl.BlockSpec((None, N, E), lambda b: (b, 0, 0)),
            pl.BlockSpec((E,), lambda b: (0,)),
            pl.BlockSpec((E,), lambda b: (0,)),
        ],
        out_specs=pl.BlockSpec((None, 1, 2 * E), lambda b: (b, 0, 0)),
        out_shape=jax.ShapeDtypeStruct((B, 1, 2 * E), jnp.float32),
    )(z, scale, shift).reshape(B, 2 * E)


def _bn_rows(h, g, b):
    m = jnp.mean(h, axis=0)
    d = h - m[None, :]
    v = jnp.mean(d * d, axis=0)
    return g[None, :] * d / jnp.sqrt(v + 1e-5)[None, :] + b[None, :]


def _mm(a, b):
    return lax.dot_general(a.astype(jnp.bfloat16), b.astype(jnp.bfloat16),
                           (((1,), (0,)), ((), ())),
                           preferred_element_type=jnp.float32)


def _s6_body(f_ref, wl1_ref, g6_ref, b6_ref, wl2_ref, bl2_ref, g7_ref,
             b7_ref, wl3_ref, bl3_ref, o_ref):
    h = _mm(f_ref[...], wl1_ref[...])
    h = _lr(_bn_rows(h, g6_ref[...], b6_ref[...]))
    h = _mm(h, wl2_ref[...]) + bl2_ref[...][None, :]
    h = _lr(_bn_rows(h, g7_ref[...], b7_ref[...]))
    o_ref[...] = _mm(h, wl3_ref[...]) + bl3_ref[...][None, :]


def _s6_call(feat, wl1, g6, b6, wl2, bl2, g7, b7, wl3, bl3):
    B = feat.shape[0]
    C = wl3.shape[1]
    return pl.pallas_call(
        _s6_body,
        out_shape=jax.ShapeDtypeStruct((B, C), jnp.float32),
    )(feat, wl1, g6, b6, wl2, bl2, g7, b7, wl3, bl3)


def _vox_edge(xt, wc, g, b, nn, k):
    cols = lax.broadcasted_iota(jnp.int32, (nn, nn), 1)
    xtx = _bdot(xt, xt)
    xx = jnp.sum(xt * xt, axis=1)
    pd = 2.0 * xtx - xx[:, None] - xx[None, :]
    pd = pd * jnp.where(xx > 0, 1.0, 1e7)[None, :]
    O = wc.shape[0]
    cmax = jnp.full((nn, O), _NEG, jnp.float32)
    cs = []
    for r in range(k):
        mx = jnp.max(pd, axis=1, keepdims=True)
        am = jnp.min(jnp.where(pd == mx, cols, nn), axis=1)
        pd = jnp.where(cols == am[:, None], _NEG, pd)
        p = (am[:, None] == cols).astype(jnp.float32)
        gth = jnp.dot(p, xt, preferred_element_type=jnp.float32,
                      precision=lax.Precision.HIGHEST)
        e = jnp.concatenate([gth - xt, xt], axis=1)
        c = _bdot(e, wc)
        cs.append(c)
        cmax = jnp.maximum(cmax, c)
    mm = sum(jnp.sum(c, axis=0) for c in cs) * (1.0 / (nn * k))
    vv = sum(jnp.sum((c - mm[None, :]) ** 2, axis=0) for c in cs) * (1.0 / (nn * k))
    return _lr(g[None, :] * (cmax - mm[None, :]) / jnp.sqrt(vv + 1e-5)[None, :] + b[None, :])


def _s7_body(v_ref, w6_ref, gc6_ref, bc6_ref, w7_ref, gc7_ref, bc7_ref,
             w8_ref, gc8_ref, bc8_ref, wl4_ref, wl5_ref, bl5_ref, o_ref):
    v = v_ref[...]
    nn = v.shape[0]
    x1 = _vox_edge(v, w6_ref[...], gc6_ref[...], bc6_ref[...], nn, _VK)
    x2 = _vox_edge(x1, w7_ref[...], gc7_ref[...], bc7_ref[...], nn, _VK)
    cat = jnp.concatenate([x1, x2], axis=1)
    c3 = _bdot(cat, w8_ref[...])
    u = _lr(_bn_rows(c3, gc8_ref[...], bc8_ref[...]))
    ff = jnp.concatenate([jnp.max(u, axis=0), jnp.sum(u, axis=0) * (1.0 / nn)])
    h = _lr(_mm(ff[None, :], wl4_ref[...]))
    o_ref[...] = _mm(h, wl5_ref[...]) + bl5_ref[...][None, :]


def _s7_call(v, w6, gc6, bc6, w7, gc7, bc7, w8, gc8, bc8, wl4, wl5, bl5):
    return pl.pallas_call(
        _s7_body,
        out_shape=jax.ShapeDtypeStruct((1, wl5.shape[1]), jnp.float32),
    )(v, w6, gc6, bc6, w7, gc7, bc7, w8, gc8, bc8, wl4, wl5, bl5)


def _pad_w(w):
    o = w.shape[0]
    return jnp.pad(w, ((0, _P - o), (0, 0))).astype(jnp.bfloat16)


def kernel(input, cloud_len_list, voxel_num, W1, W2, W3, W4, W5, g5, b5,
           Wl1, g6, b6, Wl2, bl2, g7, b7, Wl3, bl3, W6, gc6, bc6, W7, gc7,
           bc7, W8, gc8, bc8, Wl4, Wl5, bl5):
    B, N, C0 = input.shape
    x0 = jnp.pad(input, ((0, 0), (0, 0), (0, _P - C0)))

    def gather(x, idx):
        g = _gather_rows(x.reshape(B * N, _P), idx.reshape(B * _K * _N))
        return g.reshape(B, _K * N, _P)

    rn = lambda t: jnp.sum(t * t, axis=2)
    idx1 = _topk_call(x0, rn(x0))
    g0 = gather(x0, idx1)
    x1 = _conv_call(x0, g0, _pad_w(W1))
    idx2 = _topk_call(x1, rn(x1))
    g1 = gather(x1, idx2)
    x2 = _conv_call(x1, g1, _pad_w(W2))
    idx3 = _topk_call(x2, rn(x2))
    g2 = gather(x2, idx3)
    x3 = _conv_call(x2, g2, _pad_w(W3))
    idx4 = _topk_call(x3, rn(x3))
    g3 = gather(x3, idx4)

    z, ssum, ssq = _s5a_call(x3, g3, _pad_w(W4), x1, x2,
                             W5.astype(jnp.bfloat16))
    cnt = B * N
    mean = ssum / cnt
    var = ssq / cnt - mean * mean
    scale = g5 / jnp.sqrt(var + 1e-5)
    shift = b5 - mean * scale
    feat = _s5b_call(z, scale, shift)

    v = _s6_call(feat, Wl1, g6, b6, Wl2, bl2, g7, b7, Wl3, bl3)
    return _s7_call(v, W6, gc6, bc6, W7, gc7, bc7, W8, gc8, bc8, Wl4, Wl5, bl5)
